# baseline (speedup 1.0000x reference)
import functools

import jax
import jax.numpy as jnp
from jax import lax
from jax.experimental import pallas as pl
from jax.experimental.pallas import tpu as pltpu

N_ROWS = 65536
N_CODES = 1024
DIM = 64
BLOCK = 4096
GRID = N_ROWS // BLOCK


CHUNK_C = 128


def _tc_body(z_ref, cb_ref, idx_ref, loss_ref, cb2_s, csq_s):
    @pl.when(pl.program_id(0) == 0)
    def _():
        cb = cb_ref[...]
        cb2_s[...] = cb + cb
        csq_s[...] = jnp.sum(cb * cb, axis=1)[:, None]

    zbt = z_ref[...]
    zsq = jnp.sum(zbt * zbt, axis=0)
    run_min = None
    run_c = None
    for c in range(N_CODES // CHUNK_C):
        cb2c = cb2_s[pl.ds(c * CHUNK_C, CHUNK_C), :]
        csq_c = csq_s[pl.ds(c * CHUNK_C, CHUNK_C), :]
        prod2_c = lax.dot_general(cb2c, zbt, (((1,), (0,)), ((), ())),
                                  preferred_element_type=jnp.float32)
        dist_c = (zsq[None, :] + csq_c) - prod2_c
        if c == 0:
            run_min = dist_c
            run_c = jnp.zeros(dist_c.shape, jnp.int32)
        else:
            better = dist_c < run_min
            run_min = jnp.where(better, dist_c, run_min)
            run_c = jnp.where(better, c, run_c)
    jmat = run_c * CHUNK_C + lax.broadcasted_iota(jnp.int32, run_c.shape, 0)
    dmin = jnp.min(run_min, axis=0)
    idx = jnp.min(jnp.where(run_min == dmin[None, :], jmat, N_CODES),
                  axis=0)
    idx_ref[...] = idx

    @pl.when(pl.program_id(0) == 0)
    def _():
        loss_ref[...] = jnp.zeros((1, 1), jnp.float32)

    loss_ref[...] += jnp.sum(dmin).reshape(1, 1)


def _tc_argmin(zt, codebook):
    idx3, loss_sum = pl.pallas_call(
        _tc_body,
        grid=(GRID,),
        in_specs=[
            pl.BlockSpec((DIM, BLOCK), lambda i: (0, i)),
            pl.BlockSpec((N_CODES, DIM), lambda i: (0, 0)),
        ],
        out_specs=[
            pl.BlockSpec((BLOCK,), lambda i: (i,)),
            pl.BlockSpec((1, 1), lambda i: (0, 0)),
        ],
        out_shape=[
            jax.ShapeDtypeStruct((N_ROWS,), jnp.int32),
            jax.ShapeDtypeStruct((1, 1), jnp.float32),
        ],
        scratch_shapes=[
            pltpu.VMEM((N_CODES, DIM), jnp.float32),
            pltpu.VMEM((N_CODES, 1), jnp.float32),
        ],
    )(zt, codebook)
    return idx3, loss_sum[0, 0]


def _sc_gather_t(codebook_t_flat, indices):
    from jax.experimental.pallas import tpu_sc as plsc

    info = plsc.get_sparse_core_info()
    nc, ns = info.num_cores, info.num_subcores
    nw = nc * ns
    b_per_w = N_ROWS // nw
    chunk = 256
    mesh = plsc.VectorSubcoreMesh(core_axis_name="c", subcore_axis_name="s")

    n_chunks = b_per_w // chunk

    @functools.partial(
        pl.kernel, mesh=mesh,
        out_type=jax.ShapeDtypeStruct((DIM, N_ROWS), jnp.float32),
        compiler_params=pltpu.CompilerParams(needs_layout_passes=False),
        scratch_types=[
            pltpu.VMEM((DIM * N_CODES,), jnp.float32),
            pltpu.VMEM((b_per_w,), jnp.int32),
            pltpu.VMEM((2, DIM, chunk), jnp.float32),
            pltpu.SemaphoreType.DMA,
        ],
    )
    def gather_k(tab_hbm, idx_hbm, out_hbm, tab_v, idx_v, buf_v, sem):
        wid = lax.axis_index("s") * nc + lax.axis_index("c")
        base = wid * b_per_w
        pltpu.sync_copy(idx_hbm.at[pl.ds(base, b_per_w)], idx_v)
        pltpu.sync_copy(tab_hbm, tab_v)
        copies = []
        for c in range(n_chunks):
            p = c % 2
            if c >= 2:
                copies[c - 2].wait()

            @plsc.parallel_loop(0, chunk // 16, unroll=8)
            def body(g):
                iv = idx_v[pl.ds(c * chunk + g * 16, 16)]
                for d in range(DIM):
                    vals = plsc.load_gather(tab_v, [iv + d * N_CODES])
                    buf_v[p, d, pl.ds(g * 16, 16)] = vals

            copies.append(pltpu.async_copy(
                buf_v.at[p], out_hbm.at[:, pl.ds(base + c * chunk, chunk)],
                sem))
        for cp in copies[-2:]:
            cp.wait()

    return gather_k(codebook_t_flat, indices)


def kernel(z, codebook):
    indices, loss_sum = _tc_argmin(z.T, codebook)
    quantized_t = _sc_gather_t(codebook.T.reshape(-1), indices)
    vq_loss = loss_sum * jnp.float32(1.25 / (N_ROWS * DIM))
    return (quantized_t.T, indices, vq_loss)


# speedup vs baseline: 1.0521x; 1.0521x over previous
import functools

import jax
import jax.numpy as jnp
from jax import lax
from jax.experimental import pallas as pl
from jax.experimental.pallas import tpu as pltpu

N_ROWS = 65536
N_CODES = 1024
DIM = 64
BLOCK = 4096
GRID = N_ROWS // BLOCK


CHUNK_C = 128


def _tc_body(z_ref, cb_ref, idx_ref, loss_ref, cb2_s, csq_s):
    @pl.when(pl.program_id(0) == 0)
    def _():
        cb = cb_ref[...]
        cb2_s[...] = cb + cb
        csq_s[...] = jnp.sum(cb * cb, axis=1)[:, None]

    zbt = z_ref[...]
    zsq = jnp.sum(zbt * zbt, axis=0)
    run_min = None
    run_c = None
    for c in range(N_CODES // CHUNK_C):
        cb2c = cb2_s[pl.ds(c * CHUNK_C, CHUNK_C), :]
        csq_c = csq_s[pl.ds(c * CHUNK_C, CHUNK_C), :]
        prod2_c = lax.dot_general(cb2c, zbt, (((1,), (0,)), ((), ())),
                                  preferred_element_type=jnp.float32)
        dist_c = (zsq[None, :] + csq_c) - prod2_c
        if c == 0:
            run_min = dist_c
            run_c = jnp.zeros(dist_c.shape, jnp.int32)
        else:
            better = dist_c < run_min
            run_min = jnp.where(better, dist_c, run_min)
            run_c = jnp.where(better, c, run_c)
    jmat = run_c * CHUNK_C + lax.broadcasted_iota(jnp.int32, run_c.shape, 0)
    dmin = jnp.min(run_min, axis=0)
    idx = jnp.min(jnp.where(run_min == dmin[None, :], jmat, N_CODES),
                  axis=0)
    idx_ref[...] = idx

    @pl.when(pl.program_id(0) == 0)
    def _():
        loss_ref[...] = jnp.zeros((1, 1), jnp.float32)

    loss_ref[...] += jnp.sum(dmin).reshape(1, 1)


def _tc_argmin(zt, codebook):
    idx3, loss_sum = pl.pallas_call(
        _tc_body,
        grid=(GRID,),
        in_specs=[
            pl.BlockSpec((DIM, BLOCK), lambda i: (0, i)),
            pl.BlockSpec((N_CODES, DIM), lambda i: (0, 0)),
        ],
        out_specs=[
            pl.BlockSpec((BLOCK,), lambda i: (i,)),
            pl.BlockSpec((1, 1), lambda i: (0, 0)),
        ],
        out_shape=[
            jax.ShapeDtypeStruct((N_ROWS,), jnp.int32),
            jax.ShapeDtypeStruct((1, 1), jnp.float32),
        ],
        scratch_shapes=[
            pltpu.VMEM((N_CODES, DIM), jnp.float32),
            pltpu.VMEM((N_CODES, 1), jnp.float32),
        ],
    )(zt, codebook)
    return idx3, loss_sum[0, 0]


def _sc_gather_t(codebook_t_flat, indices):
    from jax.experimental.pallas import tpu_sc as plsc

    info = plsc.get_sparse_core_info()
    nc, ns = info.num_cores, info.num_subcores
    nw = nc * ns
    b_per_w = N_ROWS // nw
    chunk = 256
    mesh = plsc.VectorSubcoreMesh(core_axis_name="c", subcore_axis_name="s")

    n_chunks = b_per_w // chunk

    @functools.partial(
        pl.kernel, mesh=mesh,
        out_type=jax.ShapeDtypeStruct((DIM, N_ROWS), jnp.float32),
        compiler_params=pltpu.CompilerParams(needs_layout_passes=False),
        scratch_types=[
            pltpu.VMEM((DIM * N_CODES,), jnp.float32),
            pltpu.VMEM((b_per_w,), jnp.int32),
            pltpu.VMEM((2, DIM, chunk), jnp.float32),
            pltpu.SemaphoreType.DMA,
        ],
    )
    def gather_k(tab_hbm, idx_hbm, out_hbm, tab_v, idx_v, buf_v, sem):
        wid = lax.axis_index("s") * nc + lax.axis_index("c")
        base = wid * b_per_w
        pltpu.sync_copy(idx_hbm.at[pl.ds(base, b_per_w)], idx_v)
        pltpu.sync_copy(tab_hbm, tab_v)
        copies = []
        for c in range(n_chunks):
            p = c % 2
            if c >= 2:
                copies[c - 2].wait()

            @plsc.parallel_loop(0, chunk // 16, unroll=4)
            def body(g):
                iv = idx_v[pl.ds(c * chunk + g * 16, 16)]
                for d in range(DIM):
                    vals = plsc.load_gather(tab_v, [iv + d * N_CODES])
                    buf_v[p, d, pl.ds(g * 16, 16)] = vals

            copies.append(pltpu.async_copy(
                buf_v.at[p], out_hbm.at[:, pl.ds(base + c * chunk, chunk)],
                sem))
        for cp in copies[-2:]:
            cp.wait()

    return gather_k(codebook_t_flat, indices)


def kernel(z, codebook):
    indices, loss_sum = _tc_argmin(z.T, codebook)
    quantized_t = _sc_gather_t(codebook.T.reshape(-1), indices)
    vq_loss = loss_sum * jnp.float32(1.25 / (N_ROWS * DIM))
    return (quantized_t.T, indices, vq_loss)
